# baseline (speedup 1.0000x reference)
import functools

import jax
import jax.numpy as jnp
from jax.experimental import pallas as pl
from jax.experimental.pallas import tpu as pltpu


def _mil_body(inv_ref, x_ref, p_ref, n_ref, out_ref, acc_ref):
    b = pl.program_id(0)
    s = pl.program_id(1)
    S = pl.num_programs(1)

    m = (p_ref[0] > 0.5) & (n_ref[0] > 0.5)
    mf = m.astype(jnp.float32)
    probs = jax.nn.sigmoid(x_ref[0])
    ps = jnp.sum(probs * mf)
    pc = jnp.sum(mf)

    @pl.when(s == 0)
    def _():
        acc_ref[0] = 0.0
        acc_ref[1] = 0.0

    acc_ref[0] += ps
    acc_ref[1] += pc

    @pl.when(jnp.logical_and(b == 0, s == 0))
    def _():
        out_ref[...] = jnp.zeros_like(out_ref)

    @pl.when(s == S - 1)
    def _():
        p = acc_ref[0] / acc_ref[1]
        inv = inv_ref[b]
        term = -inv * jnp.log(p) - (1.0 - inv) * jnp.log(1.0 - p)
        out_ref[...] = out_ref[...] + term


def kernel(cancer_logits, prostate_mask, needle_mask, involvement, grade_group):
    B, _, H, W = cancer_logits.shape
    S = 2
    x = cancer_logits.reshape(B, H, W)
    pm = prostate_mask.reshape(B, H, W)
    nm = needle_mask.reshape(B, H, W)

    img_spec = pl.BlockSpec((1, H // S, W), lambda b, s: (b, s, 0))
    out = pl.pallas_call(
        _mil_body,
        grid=(B, S),
        in_specs=[
            pl.BlockSpec(memory_space=pltpu.SMEM),
            img_spec,
            img_spec,
            img_spec,
        ],
        out_specs=pl.BlockSpec((1, 1), lambda b, s: (0, 0)),
        out_shape=jax.ShapeDtypeStruct((1, 1), jnp.float32),
        scratch_shapes=[pltpu.SMEM((2,), jnp.float32)],
    )(involvement, x, pm, nm)
    return out[0, 0]


# speedup vs baseline: 1.7473x; 1.7473x over previous
import functools

import jax
import jax.numpy as jnp
from jax.experimental import pallas as pl
from jax.experimental.pallas import tpu as pltpu

_NB = 4


def _mil_body(inv_ref, x_ref, p_ref, n_ref, out_ref):
    g = pl.program_id(0)

    m = (p_ref[...] > 0.5) & (n_ref[...] > 0.5)
    mf = m.astype(jnp.float32)
    probs = jax.nn.sigmoid(x_ref[...])
    mp = probs * mf

    total = jnp.float32(0.0)
    for j in range(_NB):
        ps = jnp.sum(mp[j])
        pc = jnp.sum(mf[j])
        p = ps / pc
        inv = inv_ref[g * _NB + j]
        total += -inv * jnp.log(p) - (1.0 - inv) * jnp.log(1.0 - p)

    @pl.when(g == 0)
    def _():
        out_ref[...] = jnp.zeros_like(out_ref)

    out_ref[...] = out_ref[...] + total


def kernel(cancer_logits, prostate_mask, needle_mask, involvement, grade_group):
    B, _, H, W = cancer_logits.shape
    x = cancer_logits.reshape(B, H, W)
    pm = prostate_mask.reshape(B, H, W)
    nm = needle_mask.reshape(B, H, W)

    img_spec = pl.BlockSpec((_NB, H, W), lambda g: (g, 0, 0))
    out = pl.pallas_call(
        _mil_body,
        grid=(B // _NB,),
        in_specs=[
            pl.BlockSpec(memory_space=pltpu.SMEM),
            img_spec,
            img_spec,
            img_spec,
        ],
        out_specs=pl.BlockSpec((1, 1), lambda g: (0, 0)),
        out_shape=jax.ShapeDtypeStruct((1, 1), jnp.float32),
    )(involvement, x, pm, nm)
    return out[0, 0]
